# baseline (speedup 1.0000x reference)
import functools

import jax
import jax.numpy as jnp
from jax import lax
from jax.experimental import pallas as pl
from jax.experimental.pallas import tpu as pltpu
from jax.experimental.pallas import tpu_sc as plsc

B = 16384
D = 64
PD = 8


def _sc_gather(P_table, Q_table, uidx, pidx):
    info = plsc.get_sparse_core_info()
    NC, NS, L = info.num_cores, info.num_subcores, info.num_lanes
    NW = NC * NS
    bpw = B // NW
    mesh = plsc.VectorSubcoreMesh(core_axis_name="c", subcore_axis_name="s")

    u2 = uidx.reshape(NW, bpw)
    p2 = pidx.reshape(NW, bpw)

    @functools.partial(
        pl.kernel,
        mesh=mesh,
        out_type=[
            jax.ShapeDtypeStruct((B, D), jnp.float32),
            jax.ShapeDtypeStruct((B, D), jnp.float32),
        ],
        scratch_types=[
            pltpu.VMEM((bpw,), jnp.int32),
            pltpu.VMEM((bpw,), jnp.int32),
            pltpu.SemaphoreType.DMA,
        ],
    )
    def k(P_hbm, Q_hbm, u_hbm, pr_hbm, Pout, Qout, uv, pv, sem):
        wid = lax.axis_index("s") * NC + lax.axis_index("c")
        base = wid * bpw
        pltpu.sync_copy(u_hbm.at[wid], uv)
        pltpu.sync_copy(pr_hbm.at[wid], pv)

        def issue16(i):
            uvec = uv[pl.ds(i * L, L)]
            pvec = pv[pl.ds(i * L, L)]
            for l in range(L):
                row = base + i * L + l
                pltpu.async_copy(
                    P_hbm.at[pl.ds(uvec[l], 1)], Pout.at[pl.ds(row, 1)], sem
                )
                pltpu.async_copy(
                    Q_hbm.at[pl.ds(pvec[l], 1)], Qout.at[pl.ds(row, 1)], sem
                )

        def wait16():
            for _l in range(L):
                pltpu.make_async_copy(
                    P_hbm.at[pl.ds(0, 1)], Pout.at[pl.ds(0, 1)], sem
                ).wait()
                pltpu.make_async_copy(
                    Q_hbm.at[pl.ds(0, 1)], Qout.at[pl.ds(0, 1)], sem
                ).wait()

        n_it = bpw // L

        def prologue(i, _):
            issue16(i)
            return 0

        def steady(i, _):
            issue16(i)
            wait16()
            return 0

        def drain(i, _):
            wait16()
            return 0

        lax.fori_loop(0, PD, prologue, 0)
        lax.fori_loop(PD, n_it, steady, 0)
        lax.fori_loop(0, PD, drain, 0)

    return k(P_table, Q_table, u2, p2)


def _mlp_body(p, q, w1a, w1b, b1, w2, b2, o):
    h = jnp.dot(p[...], w1a[...], preferred_element_type=jnp.float32)
    h = h + jnp.dot(q[...], w1b[...], preferred_element_type=jnp.float32)
    h = jnp.maximum(h + b1[...], 0.0)
    o[...] = jnp.sum(h * w2[...], axis=1, keepdims=True) + b2[...]


def _tc_mlp(P, Q, W1a, W1b, b1r, w2r, b2r):
    TB = 2048
    return pl.pallas_call(
        _mlp_body,
        grid=(B // TB,),
        in_specs=[
            pl.BlockSpec((TB, D), lambda i: (i, 0)),
            pl.BlockSpec((TB, D), lambda i: (i, 0)),
            pl.BlockSpec((D, D), lambda i: (0, 0)),
            pl.BlockSpec((D, D), lambda i: (0, 0)),
            pl.BlockSpec((1, D), lambda i: (0, 0)),
            pl.BlockSpec((1, D), lambda i: (0, 0)),
            pl.BlockSpec((1, 1), lambda i: (0, 0)),
        ],
        out_specs=pl.BlockSpec((TB, 1), lambda i: (i, 0)),
        out_shape=jax.ShapeDtypeStruct((B, 1), jnp.float32),
    )(P, Q, W1a, W1b, b1r, w2r, b2r)


def kernel(user, product, P_table, Q_table, W1, b1, W2, b2):
    user = user.astype(jnp.int32)
    product = product.astype(jnp.int32)
    P, Q = _sc_gather(P_table, Q_table, user, product)
    W1a = W1[:D]
    W1b = W1[D:]
    return _tc_mlp(
        P,
        Q,
        W1a,
        W1b,
        b1.reshape(1, D),
        W2.reshape(1, D),
        b2.reshape(1, 1),
    )


# speedup vs baseline: 1.4687x; 1.4687x over previous
import functools

import jax
import jax.numpy as jnp
from jax import lax
from jax.experimental import pallas as pl
from jax.experimental.pallas import tpu as pltpu
from jax.experimental.pallas import tpu_sc as plsc

B = 16384
D = 64
CH = 128


def _sc_gather(Pp, Qp, uidx, pidx):
    info = plsc.get_sparse_core_info()
    NC, NS, L = info.num_cores, info.num_subcores, info.num_lanes
    NW = NC * NS
    bpw = B // NW
    nch = bpw // CH
    mesh = plsc.VectorSubcoreMesh(core_axis_name="c", subcore_axis_name="s")

    u3 = uidx.reshape(NW, nch, CH)
    p3 = pidx.reshape(NW, nch, CH)

    @functools.partial(
        pl.kernel,
        mesh=mesh,
        out_type=[
            jax.ShapeDtypeStruct((B, 2 * D), jnp.float32),
            jax.ShapeDtypeStruct((B, 2 * D), jnp.float32),
        ],
        scratch_types=[
            pltpu.VMEM((nch, CH), jnp.int32),
            pltpu.VMEM((nch, CH), jnp.int32),
            pltpu.VMEM((bpw, 2 * D), jnp.float32),
            pltpu.SemaphoreType.DMA,
        ],
    )
    def k(P_hbm, Q_hbm, u_hbm, pr_hbm, Pout, Qout, uv, pv, buf, sem):
        wid = lax.axis_index("s") * NC + lax.axis_index("c")
        base = wid * bpw
        pltpu.sync_copy(u_hbm.at[wid], uv)
        pltpu.sync_copy(pr_hbm.at[wid], pv)
        for idx, out in ((uv, Pout), (pv, Qout)):
            copies = []
            for c in range(nch):
                copies.append(
                    pltpu.async_copy(
                        P_hbm.at[idx.at[c]] if out is Pout else Q_hbm.at[idx.at[c]],
                        buf.at[pl.ds(c * CH, CH)],
                        sem,
                    )
                )
            for cp in copies:
                cp.wait()
            pltpu.sync_copy(buf, out.at[pl.ds(base, bpw)])

    return k(Pp, Qp, u3, p3)


def _mlp_body(p, q, w1a, w1b, b1, w2, b2, o):
    h = jnp.dot(p[...], w1a[...], preferred_element_type=jnp.float32)
    h = h + jnp.dot(q[...], w1b[...], preferred_element_type=jnp.float32)
    h = jnp.maximum(h + b1[...], 0.0)
    o[...] = jnp.sum(h * w2[...], axis=1, keepdims=True) + b2[...]


def _tc_mlp(P, Q, W1a, W1b, b1r, w2r, b2r):
    TB = 2048
    return pl.pallas_call(
        _mlp_body,
        grid=(B // TB,),
        in_specs=[
            pl.BlockSpec((TB, 2 * D), lambda i: (i, 0)),
            pl.BlockSpec((TB, 2 * D), lambda i: (i, 0)),
            pl.BlockSpec((2 * D, D), lambda i: (0, 0)),
            pl.BlockSpec((2 * D, D), lambda i: (0, 0)),
            pl.BlockSpec((1, D), lambda i: (0, 0)),
            pl.BlockSpec((1, D), lambda i: (0, 0)),
            pl.BlockSpec((1, 1), lambda i: (0, 0)),
        ],
        out_specs=pl.BlockSpec((TB, 1), lambda i: (i, 0)),
        out_shape=jax.ShapeDtypeStruct((B, 1), jnp.float32),
    )(P, Q, W1a, W1b, b1r, w2r, b2r)


def kernel(user, product, P_table, Q_table, W1, b1, W2, b2):
    user = user.astype(jnp.int32)
    product = product.astype(jnp.int32)
    Pp = jnp.pad(P_table, ((0, 0), (0, D)))
    Qp = jnp.pad(Q_table, ((0, 0), (0, D)))
    P, Q = _sc_gather(Pp, Qp, user, product)
    Z = jnp.zeros((D, D), jnp.float32)
    W1a = jnp.concatenate([W1[:D], Z], axis=0)
    W1b = jnp.concatenate([W1[D:], Z], axis=0)
    return _tc_mlp(
        P,
        Q,
        W1a,
        W1b,
        b1.reshape(1, D),
        W2.reshape(1, D),
        b2.reshape(1, 1),
    )


# speedup vs baseline: 1.4712x; 1.0017x over previous
import functools

import jax
import jax.numpy as jnp
from jax import lax
from jax.experimental import pallas as pl
from jax.experimental.pallas import tpu as pltpu
from jax.experimental.pallas import tpu_sc as plsc

B = 16384
D = 64
CH = 128


def _sc_gather(Pp3, Qp3, uidx, pidx):
    info = plsc.get_sparse_core_info()
    NC, NS, L = info.num_cores, info.num_subcores, info.num_lanes
    NW = NC * NS
    bpw = B // NW
    nch = bpw // CH
    NP = Pp3.shape[0] * 8
    NQ = Qp3.shape[0] * 8
    mesh = plsc.VectorSubcoreMesh(core_axis_name="c", subcore_axis_name="s")

    u3 = uidx.reshape(NW, nch, CH)
    p3 = pidx.reshape(NW, nch, CH)

    @functools.partial(
        pl.kernel,
        mesh=mesh,
        out_type=[
            jax.ShapeDtypeStruct((B // 8, 8, 2 * D), jnp.float32),
            jax.ShapeDtypeStruct((B // 8, 8, 2 * D), jnp.float32),
        ],
        scratch_types=[
            pltpu.VMEM((nch, CH), jnp.int32),
            pltpu.VMEM((nch, CH), jnp.int32),
            pltpu.VMEM((bpw, 2 * D), jnp.float32),
            pltpu.SemaphoreType.DMA,
        ],
    )
    def k(P_hbm, Q_hbm, u_hbm, pr_hbm, Pout, Qout, uv, pv, buf, sem):
        wid = lax.axis_index("s") * NC + lax.axis_index("c")
        base = wid * bpw
        pltpu.sync_copy(u_hbm.at[wid], uv)
        pltpu.sync_copy(pr_hbm.at[wid], pv)
        Pv = P_hbm.reshape(NP, 2 * D)
        Qv = Q_hbm.reshape(NQ, 2 * D)
        for idx, src, out in ((uv, Pv, Pout), (pv, Qv, Qout)):
            copies = []
            for c in range(nch):
                copies.append(
                    pltpu.async_copy(
                        src.at[idx.at[c]], buf.at[pl.ds(c * CH, CH)], sem
                    )
                )
            for cp in copies:
                cp.wait()
            pltpu.sync_copy(buf, out.reshape(B, 2 * D).at[pl.ds(base, bpw)])

    return k(Pp3, Qp3, u3, p3)


def _mlp_body(p, q, w1a, w1b, b1, w2, b2, o):
    pm = p[...].reshape(-1, 2 * D)
    qm = q[...].reshape(-1, 2 * D)
    h = jnp.dot(pm, w1a[...], preferred_element_type=jnp.float32)
    h = h + jnp.dot(qm, w1b[...], preferred_element_type=jnp.float32)
    h = jnp.maximum(h + b1[...], 0.0)
    o[...] = jnp.sum(h * w2[...], axis=1, keepdims=True) + b2[...]


def _tc_mlp(P3, Q3, W1a, W1b, b1r, w2r, b2r):
    TB = 2048
    return pl.pallas_call(
        _mlp_body,
        grid=(B // TB,),
        in_specs=[
            pl.BlockSpec((TB // 8, 8, 2 * D), lambda i: (i, 0, 0)),
            pl.BlockSpec((TB // 8, 8, 2 * D), lambda i: (i, 0, 0)),
            pl.BlockSpec((2 * D, D), lambda i: (0, 0)),
            pl.BlockSpec((2 * D, D), lambda i: (0, 0)),
            pl.BlockSpec((1, D), lambda i: (0, 0)),
            pl.BlockSpec((1, D), lambda i: (0, 0)),
            pl.BlockSpec((1, 1), lambda i: (0, 0)),
        ],
        out_specs=pl.BlockSpec((TB, 1), lambda i: (i, 0)),
        out_shape=jax.ShapeDtypeStruct((B, 1), jnp.float32),
    )(P3, Q3, W1a, W1b, b1r, w2r, b2r)


def kernel(user, product, P_table, Q_table, W1, b1, W2, b2):
    user = user.astype(jnp.int32)
    product = product.astype(jnp.int32)
    Pp3 = jnp.pad(P_table.reshape(-1, 8, D), ((0, 0), (0, 0), (0, D)))
    Qp3 = jnp.pad(Q_table.reshape(-1, 8, D), ((0, 0), (0, 0), (0, D)))
    P3, Q3 = _sc_gather(Pp3, Qp3, user, product)
    Z = jnp.zeros((D, D), jnp.float32)
    W1a = jnp.concatenate([W1[:D], Z], axis=0)
    W1b = jnp.concatenate([W1[D:], Z], axis=0)
    return _tc_mlp(
        P3,
        Q3,
        W1a,
        W1b,
        b1.reshape(1, D),
        W2.reshape(1, D),
        b2.reshape(1, 1),
    )
